# Initial kernel scaffold; baseline (speedup 1.0000x reference)
import jax
import jax.numpy as jnp
from jax.experimental import pallas as pl


def kernel(x, edge_index, W1, as1, ad1, b1, W2, as2, ad2, b2, W3, as3, ad3, b3, Wp, bp, Wl1, bl1, Wr1, Wl2, bl2, Wr2, Wl3, bl3, Wr3):
    raise NotImplementedError("write your pallas kernel here")


# speedup vs baseline: 1.0710x; 1.0710x over previous
import functools
import jax
import jax.numpy as jnp
from jax.experimental import pallas as pl
from jax.experimental.pallas import tpu as pltpu

_N = 10000
_E = 160000
_BLK = 1000


def _proj_first_body(x_ref, wl_ref, wr_ref, bl_ref, y_ref, z_ref):
    x = x_ref[...]
    y_ref[...] = jnp.dot(x, wl_ref[...], preferred_element_type=jnp.float32)
    z_ref[...] = (
        jnp.dot(x, wr_ref[...], preferred_element_type=jnp.float32) + bl_ref[...]
    )


def _proj_first(x, wl, wr, bl):
    n, din = x.shape
    cout = wl.shape[1]
    grid = (n // _BLK,)
    return pl.pallas_call(
        _proj_first_body,
        grid=grid,
        in_specs=[
            pl.BlockSpec((_BLK, din), lambda i: (i, 0)),
            pl.BlockSpec((din, cout), lambda i: (0, 0)),
            pl.BlockSpec((din, cout), lambda i: (0, 0)),
            pl.BlockSpec((1, cout), lambda i: (0, 0)),
        ],
        out_specs=[
            pl.BlockSpec((_BLK, cout), lambda i: (i, 0)),
            pl.BlockSpec((_BLK, cout), lambda i: (i, 0)),
        ],
        out_shape=[
            jax.ShapeDtypeStruct((n, cout), jnp.float32),
            jax.ShapeDtypeStruct((n, cout), jnp.float32),
        ],
    )(x, wl, wr, bl.reshape(1, -1))


def _proj_mid_body(agg_ref, invd_ref, z_ref, wl_ref, wr_ref, bl_ref, y_ref, z2_ref):
    xn = jnp.maximum(agg_ref[...] * invd_ref[...] + z_ref[...], 0.0)
    y_ref[...] = jnp.dot(xn, wl_ref[...], preferred_element_type=jnp.float32)
    z2_ref[...] = (
        jnp.dot(xn, wr_ref[...], preferred_element_type=jnp.float32) + bl_ref[...]
    )


def _proj_mid(agg, invd, z, wl, wr, bl):
    n, din = agg.shape
    cout = wl.shape[1]
    grid = (n // _BLK,)
    return pl.pallas_call(
        _proj_mid_body,
        grid=grid,
        in_specs=[
            pl.BlockSpec((_BLK, din), lambda i: (i, 0)),
            pl.BlockSpec((_BLK, 1), lambda i: (i, 0)),
            pl.BlockSpec((_BLK, din), lambda i: (i, 0)),
            pl.BlockSpec((din, cout), lambda i: (0, 0)),
            pl.BlockSpec((din, cout), lambda i: (0, 0)),
            pl.BlockSpec((1, cout), lambda i: (0, 0)),
        ],
        out_specs=[
            pl.BlockSpec((_BLK, cout), lambda i: (i, 0)),
            pl.BlockSpec((_BLK, cout), lambda i: (i, 0)),
        ],
        out_shape=[
            jax.ShapeDtypeStruct((n, cout), jnp.float32),
            jax.ShapeDtypeStruct((n, cout), jnp.float32),
        ],
    )(agg, invd, z, wl, wr, bl.reshape(1, -1))


def _final_body(agg_ref, invd_ref, z_ref, out_ref):
    out_ref[...] = jnp.maximum(agg_ref[...] * invd_ref[...] + z_ref[...], 0.0)


def _final(agg, invd, z):
    n, c = agg.shape
    grid = (n // _BLK,)
    return pl.pallas_call(
        _final_body,
        grid=grid,
        in_specs=[
            pl.BlockSpec((_BLK, c), lambda i: (i, 0)),
            pl.BlockSpec((_BLK, 1), lambda i: (i, 0)),
            pl.BlockSpec((_BLK, c), lambda i: (i, 0)),
        ],
        out_specs=pl.BlockSpec((_BLK, c), lambda i: (i, 0)),
        out_shape=jax.ShapeDtypeStruct((n, c), jnp.float32),
    )(agg, invd, z)


def kernel(x, edge_index, W1, as1, ad1, b1, W2, as2, ad2, b2, W3, as3, ad3, b3,
           Wp, bp, Wl1, bl1, Wr1, Wl2, bl2, Wr2, Wl3, bl3, Wr3):
    src = edge_index[0]
    dst = edge_index[1]
    cnt = jax.ops.segment_sum(jnp.ones((_E,), jnp.float32), dst, num_segments=_N)
    invd = (1.0 / jnp.clip(cnt, 1.0))[:, None]

    y1, z1 = _proj_first(x, Wl1, Wr1, bl1)
    agg1 = jax.ops.segment_sum(y1[src], dst, num_segments=_N)
    y2, z2 = _proj_mid(agg1, invd, z1, Wl2, Wr2, bl2)
    agg2 = jax.ops.segment_sum(y2[src], dst, num_segments=_N)
    y3, z3 = _proj_mid(agg2, invd, z2, Wl3, Wr3, bl3)
    agg3 = jax.ops.segment_sum(y3[src], dst, num_segments=_N)
    return _final(agg3, invd, z3)


# speedup vs baseline: 7.6071x; 7.1031x over previous
import functools
import jax
import jax.numpy as jnp
from jax import lax
from jax.experimental import pallas as pl
from jax.experimental.pallas import tpu as pltpu
from jax.experimental.pallas import tpu_sc as plsc

_N = 10000
_E = 160000
_BLK = 1024

_NC = 2
_NS = 16
_NW = _NC * _NS
_NPAD = 10240
_EPAD = 163840
_CHUNK = 128
_NCHUNKS = _EPAD // (_NW * _CHUNK)
_RPT = _NPAD // _NS


def _sc_agg_body(with_cnt, c_dim, y_hbm, srcp_hbm, dstp_hbm, zrow_hbm, zvec_hbm,
                 ones_hbm, *refs):
    if with_cnt:
        (tab_out, cnt_out, src_v, dst_v, rows_v, ones_v,
         table_sh, cnt_sh, gsem, ssem, csem) = refs
    else:
        (tab_out, src_v, dst_v, rows_v, table_sh, gsem, ssem) = refs
    cid = lax.axis_index("c")
    sid = lax.axis_index("s")
    w = cid * _NS + sid

    pltpu.sync_copy(srcp_hbm.at[w], src_v)
    pltpu.sync_copy(dstp_hbm.at[w], dst_v)

    pltpu.sync_copy(zrow_hbm, table_sh.at[pl.ds(sid * _RPT, _RPT)])
    if with_cnt:
        pltpu.sync_copy(ones_hbm, ones_v)
        pltpu.sync_copy(zvec_hbm, cnt_sh.at[pl.ds(sid * _RPT, _RPT)])
    plsc.subcore_barrier()

    def chunk(i, carry):
        pltpu.async_copy(y_hbm.at[src_v.at[i]], rows_v, gsem).wait()
        cp = pltpu.async_copy(rows_v, table_sh.at[dst_v.at[i]], ssem, add=True)
        if with_cnt:
            pltpu.async_copy(ones_v, cnt_sh.at[dst_v.at[i]], csem, add=True).wait()
        cp.wait()
        return carry

    lax.fori_loop(0, _NCHUNKS, chunk, 0)
    plsc.subcore_barrier()

    r0 = sid * _RPT
    pltpu.sync_copy(table_sh.at[pl.ds(r0, _RPT)], tab_out.at[cid, pl.ds(r0, _RPT)])
    if with_cnt:
        pltpu.sync_copy(cnt_sh.at[pl.ds(r0, _RPT)], cnt_out.at[cid, pl.ds(r0, _RPT)])


@functools.lru_cache(maxsize=None)
def _make_sc_agg(c_dim, with_cnt):
    mesh = plsc.VectorSubcoreMesh(core_axis_name="c", subcore_axis_name="s")
    out_type = [jax.ShapeDtypeStruct((_NC, _NPAD, c_dim), jnp.float32)]
    scratch = [
        pltpu.VMEM((_NCHUNKS, _CHUNK), jnp.int32),
        pltpu.VMEM((_NCHUNKS, _CHUNK), jnp.int32),
        pltpu.VMEM((_CHUNK, c_dim), jnp.float32),
        pltpu.VMEM_SHARED((_NPAD, c_dim), jnp.float32),
        pltpu.SemaphoreType.DMA,
        pltpu.SemaphoreType.DMA,
    ]
    if with_cnt:
        out_type.append(jax.ShapeDtypeStruct((_NC, _NPAD), jnp.float32))
        scratch = [
            pltpu.VMEM((_NCHUNKS, _CHUNK), jnp.int32),
            pltpu.VMEM((_NCHUNKS, _CHUNK), jnp.int32),
            pltpu.VMEM((_CHUNK, c_dim), jnp.float32),
            pltpu.VMEM((_CHUNK,), jnp.float32),
            pltpu.VMEM_SHARED((_NPAD, c_dim), jnp.float32),
            pltpu.VMEM_SHARED((_NPAD,), jnp.float32),
            pltpu.SemaphoreType.DMA,
            pltpu.SemaphoreType.DMA,
            pltpu.SemaphoreType.DMA,
        ]
    params = None
    if c_dim % 128 != 0:
        params = pltpu.CompilerParams(use_tc_tiling_on_sc=False)
    return pl.kernel(
        functools.partial(_sc_agg_body, with_cnt, c_dim),
        out_type=out_type,
        mesh=mesh,
        scratch_types=scratch,
        compiler_params=params,
    )


def _sc_agg(y, srcp, dstp, with_cnt=False):
    c_dim = y.shape[1]
    k = _make_sc_agg(c_dim, with_cnt)
    zrow = jnp.zeros((_RPT, c_dim), jnp.float32)
    zvec = jnp.zeros((_RPT,), jnp.float32)
    ones = jnp.ones((_CHUNK,), jnp.float32)
    return k(y, srcp, dstp, zrow, zvec, ones)


def _proj_first_body(x_ref, wl_ref, wr_ref, bl_ref, y_ref, z_ref):
    x = x_ref[...]
    y_ref[...] = jnp.dot(x, wl_ref[...], preferred_element_type=jnp.float32)
    z_ref[...] = (
        jnp.dot(x, wr_ref[...], preferred_element_type=jnp.float32) + bl_ref[...]
    )


def _proj_first(x, wl, wr, bl):
    n, din = x.shape
    cout = wl.shape[1]
    return pl.pallas_call(
        _proj_first_body,
        grid=(n // _BLK,),
        in_specs=[
            pl.BlockSpec((_BLK, din), lambda i: (i, 0)),
            pl.BlockSpec((din, cout), lambda i: (0, 0)),
            pl.BlockSpec((din, cout), lambda i: (0, 0)),
            pl.BlockSpec((1, cout), lambda i: (0, 0)),
        ],
        out_specs=[
            pl.BlockSpec((_BLK, cout), lambda i: (i, 0)),
            pl.BlockSpec((_BLK, cout), lambda i: (i, 0)),
        ],
        out_shape=[
            jax.ShapeDtypeStruct((n, cout), jnp.float32),
            jax.ShapeDtypeStruct((n, cout), jnp.float32),
        ],
    )(x, wl, wr, bl.reshape(1, -1))


def _proj_mid_body(t_ref, cnt_ref, z_ref, wl_ref, wr_ref, bl_ref, y_ref, z2_ref):
    i = pl.program_id(0)
    cs = cnt_ref[0, pl.ds(i * _BLK, _BLK)] + cnt_ref[1, pl.ds(i * _BLK, _BLK)]
    invd = 1.0 / jnp.maximum(cs, 1.0)
    agg = t_ref[0] + t_ref[1]
    xn = jnp.maximum(agg * invd[:, None] + z_ref[...], 0.0)
    y_ref[...] = jnp.dot(xn, wl_ref[...], preferred_element_type=jnp.float32)
    z2_ref[...] = (
        jnp.dot(xn, wr_ref[...], preferred_element_type=jnp.float32) + bl_ref[...]
    )


def _proj_mid(t, cnt, z, wl, wr, bl):
    n = t.shape[1]
    din = t.shape[2]
    cout = wl.shape[1]
    return pl.pallas_call(
        _proj_mid_body,
        grid=(n // _BLK,),
        in_specs=[
            pl.BlockSpec((_NC, _BLK, din), lambda i: (0, i, 0)),
            pl.BlockSpec((_NC, n), lambda i: (0, 0)),
            pl.BlockSpec((_BLK, din), lambda i: (i, 0)),
            pl.BlockSpec((din, cout), lambda i: (0, 0)),
            pl.BlockSpec((din, cout), lambda i: (0, 0)),
            pl.BlockSpec((1, cout), lambda i: (0, 0)),
        ],
        out_specs=[
            pl.BlockSpec((_BLK, cout), lambda i: (i, 0)),
            pl.BlockSpec((_BLK, cout), lambda i: (i, 0)),
        ],
        out_shape=[
            jax.ShapeDtypeStruct((n, cout), jnp.float32),
            jax.ShapeDtypeStruct((n, cout), jnp.float32),
        ],
    )(t, cnt, z, wl, wr, bl.reshape(1, -1))


def _final_body(t_ref, cnt_ref, z_ref, out_ref):
    i = pl.program_id(0)
    cs = cnt_ref[0, pl.ds(i * _BLK, _BLK)] + cnt_ref[1, pl.ds(i * _BLK, _BLK)]
    invd = 1.0 / jnp.maximum(cs, 1.0)
    agg = t_ref[0] + t_ref[1]
    out_ref[...] = jnp.maximum(agg * invd[:, None] + z_ref[...], 0.0)


def _final(t, cnt, z):
    n = t.shape[1]
    c = t.shape[2]
    return pl.pallas_call(
        _final_body,
        grid=(n // _BLK,),
        in_specs=[
            pl.BlockSpec((_NC, _BLK, c), lambda i: (0, i, 0)),
            pl.BlockSpec((_NC, n), lambda i: (0, 0)),
            pl.BlockSpec((_BLK, c), lambda i: (i, 0)),
        ],
        out_specs=pl.BlockSpec((_BLK, c), lambda i: (i, 0)),
        out_shape=jax.ShapeDtypeStruct((n, c), jnp.float32),
    )(t, cnt, z)


def kernel(x, edge_index, W1, as1, ad1, b1, W2, as2, ad2, b2, W3, as3, ad3, b3,
           Wp, bp, Wl1, bl1, Wr1, Wl2, bl2, Wr2, Wl3, bl3, Wr3):
    src = edge_index[0]
    dst = edge_index[1]
    pad_idx = _N + (jnp.arange(_EPAD - _E, dtype=jnp.int32) % (_NPAD - _N))
    srcp = jnp.concatenate([src, pad_idx]).reshape(_NW, _NCHUNKS, _CHUNK)
    dstp = jnp.concatenate([dst, pad_idx]).reshape(_NW, _NCHUNKS, _CHUNK)
    x_pad = jnp.pad(x, ((0, _NPAD - _N), (0, 0)))

    y1, z1 = _proj_first(x_pad, Wl1, Wr1, bl1)
    t1, cnt = _sc_agg(y1, srcp, dstp, with_cnt=True)
    y2, z2 = _proj_mid(t1, cnt, z1, Wl2, Wr2, bl2)
    t2, = _sc_agg(y2, srcp, dstp)
    y3, z3 = _proj_mid(t2, cnt, z2, Wl3, Wr3, bl3)
    t3, = _sc_agg(y3, srcp, dstp)
    out = _final(t3, cnt, z3)
    return out[:_N]


# speedup vs baseline: 9.0626x; 1.1913x over previous
import functools
import jax
import jax.numpy as jnp
from jax import lax
from jax.experimental import pallas as pl
from jax.experimental.pallas import tpu as pltpu
from jax.experimental.pallas import tpu_sc as plsc

_N = 10000
_E = 160000
_BLK = 1024

_NC = 2
_NS = 16
_NW = _NC * _NS
_NPAD = 10240
_EPAD = 163840
_CHUNK = 128
_NCHUNKS = _EPAD // (_NW * _CHUNK)
_RPT = _NPAD // _NS


def _sc_agg_body(with_cnt, c_dim, y_hbm, srcp_hbm, dstp_hbm, zrow_hbm, zvec_hbm,
                 ones_hbm, *refs):
    if with_cnt:
        (tab_out, cnt_out, src_v, dst_v, rows_v, ones_v,
         table_sh, cnt_sh, gsems, ssems, csems) = refs
    else:
        (tab_out, src_v, dst_v, rows_v, table_sh, gsems, ssems) = refs
        csems = None
    cid = lax.axis_index("c")
    sid = lax.axis_index("s")
    w = cid * _NS + sid

    pltpu.sync_copy(srcp_hbm.at[w], src_v)
    pltpu.sync_copy(dstp_hbm.at[w], dst_v)

    pltpu.sync_copy(zrow_hbm, table_sh.at[pl.ds(sid * _RPT, _RPT)])
    if with_cnt:
        pltpu.sync_copy(ones_hbm, ones_v)
        pltpu.sync_copy(zvec_hbm, cnt_sh.at[pl.ds(sid * _RPT, _RPT)])
    plsc.subcore_barrier()

    pltpu.async_copy(y_hbm.at[src_v.at[0]], rows_v.at[0], gsems[0])

    def step(s, carry):
        for b in range(2):
            i = 2 * s + b
            nb = 1 - b
            pltpu.make_async_copy(y_hbm.at[src_v.at[i]], rows_v.at[b],
                                  gsems[b]).wait()

            @pl.when(i + 1 < _NCHUNKS)
            def _():
                @pl.when(i >= 1)
                def _():
                    pltpu.make_async_copy(
                        rows_v.at[nb], table_sh.at[dst_v.at[i]],
                        ssems[nb]).wait()
                    if with_cnt:
                        pltpu.make_async_copy(
                            ones_v, cnt_sh.at[dst_v.at[i]], csems[nb]).wait()
                pltpu.async_copy(y_hbm.at[src_v.at[i + 1]], rows_v.at[nb],
                                 gsems[nb])

            pltpu.async_copy(rows_v.at[b], table_sh.at[dst_v.at[i]], ssems[b],
                             add=True)
            if with_cnt:
                pltpu.async_copy(ones_v, cnt_sh.at[dst_v.at[i]], csems[b],
                                 add=True)
        return carry

    lax.fori_loop(0, _NCHUNKS // 2, step, 0)
    for b in range(2):
        pltpu.make_async_copy(rows_v.at[b], table_sh.at[dst_v.at[0]],
                              ssems[b]).wait()
        if with_cnt:
            pltpu.make_async_copy(ones_v, cnt_sh.at[dst_v.at[0]],
                                  csems[b]).wait()
    plsc.subcore_barrier()

    r0 = sid * _RPT
    pltpu.sync_copy(table_sh.at[pl.ds(r0, _RPT)], tab_out.at[cid, pl.ds(r0, _RPT)])
    if with_cnt:
        pltpu.sync_copy(cnt_sh.at[pl.ds(r0, _RPT)], cnt_out.at[cid, pl.ds(r0, _RPT)])


@functools.lru_cache(maxsize=None)
def _make_sc_agg(c_dim, with_cnt):
    mesh = plsc.VectorSubcoreMesh(core_axis_name="c", subcore_axis_name="s")
    out_type = [jax.ShapeDtypeStruct((_NC, _NPAD, c_dim), jnp.float32)]
    sem_pair = [pltpu.SemaphoreType.DMA, pltpu.SemaphoreType.DMA]
    scratch = [
        pltpu.VMEM((_NCHUNKS, _CHUNK), jnp.int32),
        pltpu.VMEM((_NCHUNKS, _CHUNK), jnp.int32),
        pltpu.VMEM((2, _CHUNK, c_dim), jnp.float32),
        pltpu.VMEM_SHARED((_NPAD, c_dim), jnp.float32),
        list(sem_pair),
        list(sem_pair),
    ]
    if with_cnt:
        out_type.append(jax.ShapeDtypeStruct((_NC, _NPAD), jnp.float32))
        scratch = [
            pltpu.VMEM((_NCHUNKS, _CHUNK), jnp.int32),
            pltpu.VMEM((_NCHUNKS, _CHUNK), jnp.int32),
            pltpu.VMEM((2, _CHUNK, c_dim), jnp.float32),
            pltpu.VMEM((_CHUNK,), jnp.float32),
            pltpu.VMEM_SHARED((_NPAD, c_dim), jnp.float32),
            pltpu.VMEM_SHARED((_NPAD,), jnp.float32),
            list(sem_pair),
            list(sem_pair),
            list(sem_pair),
        ]
    params = None
    if c_dim % 128 != 0:
        params = pltpu.CompilerParams(use_tc_tiling_on_sc=False)
    return pl.kernel(
        functools.partial(_sc_agg_body, with_cnt, c_dim),
        out_type=out_type,
        mesh=mesh,
        scratch_types=scratch,
        compiler_params=params,
    )


def _sc_agg(y, srcp, dstp, with_cnt=False):
    c_dim = y.shape[1]
    k = _make_sc_agg(c_dim, with_cnt)
    zrow = jnp.zeros((_RPT, c_dim), jnp.float32)
    zvec = jnp.zeros((_RPT,), jnp.float32)
    ones = jnp.ones((_CHUNK,), jnp.float32)
    return k(y, srcp, dstp, zrow, zvec, ones)


def _proj_first_body(x_ref, wl_ref, wr_ref, bl_ref, y_ref, z_ref):
    x = x_ref[...]
    y_ref[...] = jnp.dot(x, wl_ref[...], preferred_element_type=jnp.float32)
    z_ref[...] = (
        jnp.dot(x, wr_ref[...], preferred_element_type=jnp.float32) + bl_ref[...]
    )


def _proj_first(x, wl, wr, bl):
    n, din = x.shape
    cout = wl.shape[1]
    return pl.pallas_call(
        _proj_first_body,
        grid=(n // _BLK,),
        in_specs=[
            pl.BlockSpec((_BLK, din), lambda i: (i, 0)),
            pl.BlockSpec((din, cout), lambda i: (0, 0)),
            pl.BlockSpec((din, cout), lambda i: (0, 0)),
            pl.BlockSpec((1, cout), lambda i: (0, 0)),
        ],
        out_specs=[
            pl.BlockSpec((_BLK, cout), lambda i: (i, 0)),
            pl.BlockSpec((_BLK, cout), lambda i: (i, 0)),
        ],
        out_shape=[
            jax.ShapeDtypeStruct((n, cout), jnp.float32),
            jax.ShapeDtypeStruct((n, cout), jnp.float32),
        ],
    )(x, wl, wr, bl.reshape(1, -1))


def _proj_mid_body(t_ref, cnt_ref, z_ref, wl_ref, wr_ref, bl_ref, y_ref, z2_ref):
    i = pl.program_id(0)
    cs = cnt_ref[0, pl.ds(i * _BLK, _BLK)] + cnt_ref[1, pl.ds(i * _BLK, _BLK)]
    invd = 1.0 / jnp.maximum(cs, 1.0)
    agg = t_ref[0] + t_ref[1]
    xn = jnp.maximum(agg * invd[:, None] + z_ref[...], 0.0)
    y_ref[...] = jnp.dot(xn, wl_ref[...], preferred_element_type=jnp.float32)
    z2_ref[...] = (
        jnp.dot(xn, wr_ref[...], preferred_element_type=jnp.float32) + bl_ref[...]
    )


def _proj_mid(t, cnt, z, wl, wr, bl):
    n = t.shape[1]
    din = t.shape[2]
    cout = wl.shape[1]
    return pl.pallas_call(
        _proj_mid_body,
        grid=(n // _BLK,),
        in_specs=[
            pl.BlockSpec((_NC, _BLK, din), lambda i: (0, i, 0)),
            pl.BlockSpec((_NC, n), lambda i: (0, 0)),
            pl.BlockSpec((_BLK, din), lambda i: (i, 0)),
            pl.BlockSpec((din, cout), lambda i: (0, 0)),
            pl.BlockSpec((din, cout), lambda i: (0, 0)),
            pl.BlockSpec((1, cout), lambda i: (0, 0)),
        ],
        out_specs=[
            pl.BlockSpec((_BLK, cout), lambda i: (i, 0)),
            pl.BlockSpec((_BLK, cout), lambda i: (i, 0)),
        ],
        out_shape=[
            jax.ShapeDtypeStruct((n, cout), jnp.float32),
            jax.ShapeDtypeStruct((n, cout), jnp.float32),
        ],
    )(t, cnt, z, wl, wr, bl.reshape(1, -1))


def _final_body(t_ref, cnt_ref, z_ref, out_ref):
    i = pl.program_id(0)
    cs = cnt_ref[0, pl.ds(i * _BLK, _BLK)] + cnt_ref[1, pl.ds(i * _BLK, _BLK)]
    invd = 1.0 / jnp.maximum(cs, 1.0)
    agg = t_ref[0] + t_ref[1]
    out_ref[...] = jnp.maximum(agg * invd[:, None] + z_ref[...], 0.0)


def _final(t, cnt, z):
    n = t.shape[1]
    c = t.shape[2]
    return pl.pallas_call(
        _final_body,
        grid=(n // _BLK,),
        in_specs=[
            pl.BlockSpec((_NC, _BLK, c), lambda i: (0, i, 0)),
            pl.BlockSpec((_NC, n), lambda i: (0, 0)),
            pl.BlockSpec((_BLK, c), lambda i: (i, 0)),
        ],
        out_specs=pl.BlockSpec((_BLK, c), lambda i: (i, 0)),
        out_shape=jax.ShapeDtypeStruct((n, c), jnp.float32),
    )(t, cnt, z)


def kernel(x, edge_index, W1, as1, ad1, b1, W2, as2, ad2, b2, W3, as3, ad3, b3,
           Wp, bp, Wl1, bl1, Wr1, Wl2, bl2, Wr2, Wl3, bl3, Wr3):
    src = edge_index[0]
    dst = edge_index[1]
    pad_idx = _N + (jnp.arange(_EPAD - _E, dtype=jnp.int32) % (_NPAD - _N))
    srcp = jnp.concatenate([src, pad_idx]).reshape(_NW, _NCHUNKS, _CHUNK)
    dstp = jnp.concatenate([dst, pad_idx]).reshape(_NW, _NCHUNKS, _CHUNK)
    x_pad = jnp.pad(x, ((0, _NPAD - _N), (0, 0)))

    y1, z1 = _proj_first(x_pad, Wl1, Wr1, bl1)
    t1, cnt = _sc_agg(y1, srcp, dstp, with_cnt=True)
    y2, z2 = _proj_mid(t1, cnt, z1, Wl2, Wr2, bl2)
    t2, = _sc_agg(y2, srcp, dstp)
    y3, z3 = _proj_mid(t2, cnt, z2, Wl3, Wr3, bl3)
    t3, = _sc_agg(y3, srcp, dstp)
    out = _final(t3, cnt, z3)
    return out[:_N]


# speedup vs baseline: 9.0789x; 1.0018x over previous
import functools
import jax
import jax.numpy as jnp
from jax import lax
from jax.experimental import pallas as pl
from jax.experimental.pallas import tpu as pltpu
from jax.experimental.pallas import tpu_sc as plsc

_N = 10000
_E = 160000
_BLK = 1024

_NC = 2
_NS = 16
_NW = _NC * _NS
_NPAD = 10240
_CHUNK = 40
_NCHUNKS = 126
_EPAD = _NW * _NCHUNKS * _CHUNK
_RPT = _NPAD // _NS


def _sc_agg_body(with_cnt, c_dim, nbuf, k_ahead,
                 y_hbm, srcp_hbm, dstp_hbm, zrow_hbm, zvec_hbm, ones_hbm,
                 *refs):
    if with_cnt:
        (tab_out, cnt_out, src_v, dst_v, rows_v, ones_v,
         table_sh, cnt_sh, gsems, ssems, csems) = refs
    else:
        (tab_out, src_v, dst_v, rows_v, table_sh, gsems, ssems) = refs
    cid = lax.axis_index("c")
    sid = lax.axis_index("s")
    w = cid * _NS + sid

    pltpu.sync_copy(srcp_hbm.at[w], src_v)
    pltpu.sync_copy(dstp_hbm.at[w], dst_v)

    pltpu.sync_copy(zrow_hbm, table_sh.at[pl.ds(sid * _RPT, _RPT)])
    if with_cnt:
        pltpu.sync_copy(ones_hbm, ones_v)
        pltpu.sync_copy(zvec_hbm, cnt_sh.at[pl.ds(sid * _RPT, _RPT)])
    plsc.subcore_barrier()

    for p in range(k_ahead):
        pltpu.async_copy(y_hbm.at[src_v.at[p]], rows_v.at[p], gsems[p])

    def step(s, carry):
        for b in range(nbuf):
            i = nbuf * s + b
            pltpu.make_async_copy(y_hbm.at[src_v.at[i]], rows_v.at[b],
                                  gsems[b]).wait()

            bj = (b + k_ahead) % nbuf

            @pl.when(i + k_ahead < _NCHUNKS)
            def _():
                @pl.when(i + k_ahead >= nbuf)
                def _():
                    pltpu.make_async_copy(
                        rows_v.at[bj], table_sh.at[dst_v.at[i]],
                        ssems[bj]).wait()
                    if with_cnt:
                        pltpu.make_async_copy(
                            ones_v, cnt_sh.at[dst_v.at[i]], csems[bj]).wait()
                pltpu.async_copy(y_hbm.at[src_v.at[i + k_ahead]],
                                 rows_v.at[bj], gsems[bj])

            pltpu.async_copy(rows_v.at[b], table_sh.at[dst_v.at[i]], ssems[b],
                             add=True)
            if with_cnt:
                pltpu.async_copy(ones_v, cnt_sh.at[dst_v.at[i]], csems[b],
                                 add=True)
        return carry

    lax.fori_loop(0, _NCHUNKS // nbuf, step, 0)
    for b in range(nbuf):
        pltpu.make_async_copy(rows_v.at[b], table_sh.at[dst_v.at[0]],
                              ssems[b]).wait()
        if with_cnt:
            pltpu.make_async_copy(ones_v, cnt_sh.at[dst_v.at[0]],
                                  csems[b]).wait()
    plsc.subcore_barrier()

    r0 = sid * _RPT
    pltpu.sync_copy(table_sh.at[pl.ds(r0, _RPT)], tab_out.at[cid, pl.ds(r0, _RPT)])
    if with_cnt:
        pltpu.sync_copy(cnt_sh.at[pl.ds(r0, _RPT)], cnt_out.at[cid, pl.ds(r0, _RPT)])


@functools.lru_cache(maxsize=None)
def _make_sc_agg(c_dim, with_cnt, nbuf, k_ahead):
    mesh = plsc.VectorSubcoreMesh(core_axis_name="c", subcore_axis_name="s")
    out_type = [jax.ShapeDtypeStruct((_NC, _NPAD, c_dim), jnp.float32)]

    def sems():
        return [pltpu.SemaphoreType.DMA for _ in range(nbuf)]
    scratch = [
        pltpu.VMEM((_NCHUNKS, _CHUNK), jnp.int32),
        pltpu.VMEM((_NCHUNKS, _CHUNK), jnp.int32),
        pltpu.VMEM((nbuf, _CHUNK, c_dim), jnp.float32),
        pltpu.VMEM_SHARED((_NPAD, c_dim), jnp.float32),
        sems(),
        sems(),
    ]
    if with_cnt:
        out_type.append(jax.ShapeDtypeStruct((_NC, _NPAD), jnp.float32))
        scratch = [
            pltpu.VMEM((_NCHUNKS, _CHUNK), jnp.int32),
            pltpu.VMEM((_NCHUNKS, _CHUNK), jnp.int32),
            pltpu.VMEM((nbuf, _CHUNK, c_dim), jnp.float32),
            pltpu.VMEM((_CHUNK,), jnp.float32),
            pltpu.VMEM_SHARED((_NPAD, c_dim), jnp.float32),
            pltpu.VMEM_SHARED((_NPAD,), jnp.float32),
            sems(),
            sems(),
            sems(),
        ]
    params = None
    if c_dim % 128 != 0:
        params = pltpu.CompilerParams(use_tc_tiling_on_sc=False)
    return pl.kernel(
        functools.partial(_sc_agg_body, with_cnt, c_dim, nbuf, k_ahead),
        out_type=out_type,
        mesh=mesh,
        scratch_types=scratch,
        compiler_params=params,
    )


def _sc_agg(y, srcp, dstp, with_cnt=False, nbuf=3, k_ahead=2):
    c_dim = y.shape[1]
    k = _make_sc_agg(c_dim, with_cnt, nbuf, k_ahead)
    zrow = jnp.zeros((_RPT, c_dim), jnp.float32)
    zvec = jnp.zeros((_RPT,), jnp.float32)
    ones = jnp.ones((_CHUNK,), jnp.float32)
    return k(y, srcp, dstp, zrow, zvec, ones)


def _proj_first_body(x_ref, wl_ref, wr_ref, bl_ref, y_ref, z_ref):
    x = x_ref[...]
    y_ref[...] = jnp.dot(x, wl_ref[...], preferred_element_type=jnp.float32)
    z_ref[...] = (
        jnp.dot(x, wr_ref[...], preferred_element_type=jnp.float32) + bl_ref[...]
    )


def _proj_first(x, wl, wr, bl):
    n, din = x.shape
    cout = wl.shape[1]
    return pl.pallas_call(
        _proj_first_body,
        grid=(n // _BLK,),
        in_specs=[
            pl.BlockSpec((_BLK, din), lambda i: (i, 0)),
            pl.BlockSpec((din, cout), lambda i: (0, 0)),
            pl.BlockSpec((din, cout), lambda i: (0, 0)),
            pl.BlockSpec((1, cout), lambda i: (0, 0)),
        ],
        out_specs=[
            pl.BlockSpec((_BLK, cout), lambda i: (i, 0)),
            pl.BlockSpec((_BLK, cout), lambda i: (i, 0)),
        ],
        out_shape=[
            jax.ShapeDtypeStruct((n, cout), jnp.float32),
            jax.ShapeDtypeStruct((n, cout), jnp.float32),
        ],
    )(x, wl, wr, bl.reshape(1, -1))


def _proj_mid_body(t_ref, cnt_ref, z_ref, wl_ref, wr_ref, bl_ref, y_ref, z2_ref):
    i = pl.program_id(0)
    cs = cnt_ref[0, pl.ds(i * _BLK, _BLK)] + cnt_ref[1, pl.ds(i * _BLK, _BLK)]
    invd = 1.0 / jnp.maximum(cs, 1.0)
    agg = t_ref[0] + t_ref[1]
    xn = jnp.maximum(agg * invd[:, None] + z_ref[...], 0.0)
    y_ref[...] = jnp.dot(xn, wl_ref[...], preferred_element_type=jnp.float32)
    z2_ref[...] = (
        jnp.dot(xn, wr_ref[...], preferred_element_type=jnp.float32) + bl_ref[...]
    )


def _proj_mid(t, cnt, z, wl, wr, bl):
    n = t.shape[1]
    din = t.shape[2]
    cout = wl.shape[1]
    return pl.pallas_call(
        _proj_mid_body,
        grid=(n // _BLK,),
        in_specs=[
            pl.BlockSpec((_NC, _BLK, din), lambda i: (0, i, 0)),
            pl.BlockSpec((_NC, n), lambda i: (0, 0)),
            pl.BlockSpec((_BLK, din), lambda i: (i, 0)),
            pl.BlockSpec((din, cout), lambda i: (0, 0)),
            pl.BlockSpec((din, cout), lambda i: (0, 0)),
            pl.BlockSpec((1, cout), lambda i: (0, 0)),
        ],
        out_specs=[
            pl.BlockSpec((_BLK, cout), lambda i: (i, 0)),
            pl.BlockSpec((_BLK, cout), lambda i: (i, 0)),
        ],
        out_shape=[
            jax.ShapeDtypeStruct((n, cout), jnp.float32),
            jax.ShapeDtypeStruct((n, cout), jnp.float32),
        ],
    )(t, cnt, z, wl, wr, bl.reshape(1, -1))


def _final_body(t_ref, cnt_ref, z_ref, out_ref):
    i = pl.program_id(0)
    cs = cnt_ref[0, pl.ds(i * _BLK, _BLK)] + cnt_ref[1, pl.ds(i * _BLK, _BLK)]
    invd = 1.0 / jnp.maximum(cs, 1.0)
    agg = t_ref[0] + t_ref[1]
    out_ref[...] = jnp.maximum(agg * invd[:, None] + z_ref[...], 0.0)


def _final(t, cnt, z):
    n = t.shape[1]
    c = t.shape[2]
    return pl.pallas_call(
        _final_body,
        grid=(n // _BLK,),
        in_specs=[
            pl.BlockSpec((_NC, _BLK, c), lambda i: (0, i, 0)),
            pl.BlockSpec((_NC, n), lambda i: (0, 0)),
            pl.BlockSpec((_BLK, c), lambda i: (i, 0)),
        ],
        out_specs=pl.BlockSpec((_BLK, c), lambda i: (i, 0)),
        out_shape=jax.ShapeDtypeStruct((n, c), jnp.float32),
    )(t, cnt, z)


def kernel(x, edge_index, W1, as1, ad1, b1, W2, as2, ad2, b2, W3, as3, ad3, b3,
           Wp, bp, Wl1, bl1, Wr1, Wl2, bl2, Wr2, Wl3, bl3, Wr3):
    src = edge_index[0]
    dst = edge_index[1]
    pad_idx = _N + (jnp.arange(_EPAD - _E, dtype=jnp.int32) % (_NPAD - _N))
    srcp = jnp.concatenate([src, pad_idx]).reshape(_NW, _NCHUNKS, _CHUNK)
    dstp = jnp.concatenate([dst, pad_idx]).reshape(_NW, _NCHUNKS, _CHUNK)
    x_pad = jnp.pad(x, ((0, _NPAD - _N), (0, 0)))

    y1, z1 = _proj_first(x_pad, Wl1, Wr1, bl1)
    t1, cnt = _sc_agg(y1, srcp, dstp, with_cnt=True)
    y2, z2 = _proj_mid(t1, cnt, z1, Wl2, Wr2, bl2)
    t2, = _sc_agg(y2, srcp, dstp)
    y3, z3 = _proj_mid(t2, cnt, z2, Wl3, Wr3, bl3)
    t3, = _sc_agg(y3, srcp, dstp, nbuf=6, k_ahead=4)
    out = _final(t3, cnt, z3)
    return out[:_N]


# speedup vs baseline: 9.0799x; 1.0001x over previous
import functools
import jax
import jax.numpy as jnp
from jax import lax
from jax.experimental import pallas as pl
from jax.experimental.pallas import tpu as pltpu
from jax.experimental.pallas import tpu_sc as plsc

_N = 10000
_E = 160000
_BLK = 1024

_NC = 2
_NS = 16
_NW = _NC * _NS
_NPAD = 10240
_EPW = 5040
_EPAD = _NW * _EPW
_NTAB = 10240
_RPT = _NTAB // _NS


def _sc_agg_body(with_cnt, c_dim, nbuf, k_ahead, nchunks,
                 y_hbm, srcp_hbm, dstp_hbm, zrow_hbm, zvec_hbm, ones_hbm,
                 *refs):
    if with_cnt:
        (tab_out, cnt_out, src_v, dst_v, rows_v, ones_v,
         table_sh, cnt_sh, sems) = refs
    else:
        (tab_out, src_v, dst_v, rows_v, table_sh, sems) = refs
    cid = lax.axis_index("c")
    sid = lax.axis_index("s")
    w = cid * _NS + sid

    pltpu.sync_copy(srcp_hbm.at[w], src_v)
    pltpu.sync_copy(dstp_hbm.at[w], dst_v)

    pltpu.sync_copy(zrow_hbm, table_sh.at[pl.ds(sid * _RPT, _RPT)])
    if with_cnt:
        pltpu.sync_copy(ones_hbm, ones_v)
        pltpu.sync_copy(zvec_hbm, cnt_sh.at[pl.ds(sid * _RPT, _RPT)])
    plsc.subcore_barrier()

    for p in range(k_ahead):
        pltpu.async_copy(y_hbm.at[src_v.at[p]], rows_v.at[p], sems[p])

    def step(s, carry):
        for b in range(nbuf):
            i = nbuf * s + b
            pltpu.make_async_copy(y_hbm.at[src_v.at[i]], rows_v.at[b],
                                  sems[b]).wait()

            bj = (b + k_ahead) % nbuf

            @pl.when(i + k_ahead < nchunks)
            def _():
                @pl.when(i + k_ahead >= nbuf)
                def _():
                    pltpu.make_async_copy(
                        rows_v.at[bj], table_sh.at[dst_v.at[i]],
                        sems[bj]).wait()
                    if with_cnt:
                        pltpu.make_async_copy(
                            ones_v, cnt_sh.at[dst_v.at[i]], sems[bj]).wait()
                pltpu.async_copy(y_hbm.at[src_v.at[i + k_ahead]],
                                 rows_v.at[bj], sems[bj])

            pltpu.async_copy(rows_v.at[b], table_sh.at[dst_v.at[i]], sems[b],
                             add=True)
            if with_cnt:
                pltpu.async_copy(ones_v, cnt_sh.at[dst_v.at[i]], sems[b],
                                 add=True)
        return carry

    lax.fori_loop(0, nchunks // nbuf, step, 0)
    for b in range(nbuf):
        pltpu.make_async_copy(rows_v.at[b], table_sh.at[dst_v.at[0]],
                              sems[b]).wait()
        if with_cnt:
            pltpu.make_async_copy(ones_v, cnt_sh.at[dst_v.at[0]],
                                  sems[b]).wait()
    plsc.subcore_barrier()

    r0 = sid * _RPT
    pltpu.sync_copy(table_sh.at[pl.ds(r0, _RPT)], tab_out.at[cid, pl.ds(r0, _RPT)])
    if with_cnt:
        pltpu.sync_copy(cnt_sh.at[pl.ds(r0, _RPT)], cnt_out.at[cid, pl.ds(r0, _RPT)])


@functools.lru_cache(maxsize=None)
def _make_sc_agg(c_dim, with_cnt, nbuf, k_ahead, chunk):
    nchunks = _EPW // chunk
    mesh = plsc.VectorSubcoreMesh(core_axis_name="c", subcore_axis_name="s")
    out_type = [jax.ShapeDtypeStruct((_NC, _NPAD, c_dim), jnp.float32)]

    def sems():
        return [pltpu.SemaphoreType.DMA for _ in range(nbuf)]
    scratch = [
        pltpu.VMEM((nchunks, chunk), jnp.int32),
        pltpu.VMEM((nchunks, chunk), jnp.int32),
        pltpu.VMEM((nbuf, chunk, c_dim), jnp.float32),
        pltpu.VMEM_SHARED((_NTAB, c_dim), jnp.float32),
        sems(),
    ]
    if with_cnt:
        out_type.append(jax.ShapeDtypeStruct((_NC, _NPAD), jnp.float32))
        scratch = [
            pltpu.VMEM((nchunks, chunk), jnp.int32),
            pltpu.VMEM((nchunks, chunk), jnp.int32),
            pltpu.VMEM((nbuf, chunk, c_dim), jnp.float32),
            pltpu.VMEM((chunk,), jnp.float32),
            pltpu.VMEM_SHARED((_NTAB, c_dim), jnp.float32),
            pltpu.VMEM_SHARED((_NTAB,), jnp.float32),
            sems(),
        ]
    params = None
    if c_dim % 128 != 0:
        params = pltpu.CompilerParams(use_tc_tiling_on_sc=False)
    return pl.kernel(
        functools.partial(_sc_agg_body, with_cnt, c_dim, nbuf, k_ahead, nchunks),
        out_type=out_type,
        mesh=mesh,
        scratch_types=scratch,
        compiler_params=params,
    )


def _sc_agg(y, src_flat, dst_flat, with_cnt=False, nbuf=3, k_ahead=2,
            chunk=40):
    c_dim = y.shape[1]
    nchunks = _EPW // chunk
    k = _make_sc_agg(c_dim, with_cnt, nbuf, k_ahead, chunk)
    srcp = src_flat.reshape(_NW, nchunks, chunk)
    dstp = dst_flat.reshape(_NW, nchunks, chunk)
    zrow = jnp.zeros((_RPT, c_dim), jnp.float32)
    zvec = jnp.zeros((_RPT,), jnp.float32)
    ones = jnp.ones((chunk,), jnp.float32)
    return k(y, srcp, dstp, zrow, zvec, ones)


def _proj_first_body(x_ref, wl_ref, wr_ref, bl_ref, y_ref, z_ref):
    x = x_ref[...]
    y_ref[...] = jnp.dot(x, wl_ref[...], preferred_element_type=jnp.float32)
    z_ref[...] = (
        jnp.dot(x, wr_ref[...], preferred_element_type=jnp.float32) + bl_ref[...]
    )


def _proj_first(x, wl, wr, bl):
    n, din = x.shape
    cout = wl.shape[1]
    return pl.pallas_call(
        _proj_first_body,
        grid=(n // _BLK,),
        in_specs=[
            pl.BlockSpec((_BLK, din), lambda i: (i, 0)),
            pl.BlockSpec((din, cout), lambda i: (0, 0)),
            pl.BlockSpec((din, cout), lambda i: (0, 0)),
            pl.BlockSpec((1, cout), lambda i: (0, 0)),
        ],
        out_specs=[
            pl.BlockSpec((_BLK, cout), lambda i: (i, 0)),
            pl.BlockSpec((_BLK, cout), lambda i: (i, 0)),
        ],
        out_shape=[
            jax.ShapeDtypeStruct((n, cout), jnp.float32),
            jax.ShapeDtypeStruct((n, cout), jnp.float32),
        ],
    )(x, wl, wr, bl.reshape(1, -1))


def _proj_mid_body(t_ref, cnt_ref, z_ref, wl_ref, wr_ref, bl_ref, y_ref, z2_ref):
    i = pl.program_id(0)
    cs = cnt_ref[0, pl.ds(i * _BLK, _BLK)] + cnt_ref[1, pl.ds(i * _BLK, _BLK)]
    invd = 1.0 / jnp.maximum(cs, 1.0)
    agg = t_ref[0] + t_ref[1]
    xn = jnp.maximum(agg * invd[:, None] + z_ref[...], 0.0)
    y_ref[...] = jnp.dot(xn, wl_ref[...], preferred_element_type=jnp.float32)
    z2_ref[...] = (
        jnp.dot(xn, wr_ref[...], preferred_element_type=jnp.float32) + bl_ref[...]
    )


def _proj_mid(t, cnt, z, wl, wr, bl):
    n = t.shape[1]
    din = t.shape[2]
    cout = wl.shape[1]
    return pl.pallas_call(
        _proj_mid_body,
        grid=(n // _BLK,),
        in_specs=[
            pl.BlockSpec((_NC, _BLK, din), lambda i: (0, i, 0)),
            pl.BlockSpec((_NC, n), lambda i: (0, 0)),
            pl.BlockSpec((_BLK, din), lambda i: (i, 0)),
            pl.BlockSpec((din, cout), lambda i: (0, 0)),
            pl.BlockSpec((din, cout), lambda i: (0, 0)),
            pl.BlockSpec((1, cout), lambda i: (0, 0)),
        ],
        out_specs=[
            pl.BlockSpec((_BLK, cout), lambda i: (i, 0)),
            pl.BlockSpec((_BLK, cout), lambda i: (i, 0)),
        ],
        out_shape=[
            jax.ShapeDtypeStruct((n, cout), jnp.float32),
            jax.ShapeDtypeStruct((n, cout), jnp.float32),
        ],
    )(t, cnt, z, wl, wr, bl.reshape(1, -1))


def _final_body(t_ref, cnt_ref, z_ref, out_ref):
    i = pl.program_id(0)
    cs = cnt_ref[0, pl.ds(i * _BLK, _BLK)] + cnt_ref[1, pl.ds(i * _BLK, _BLK)]
    invd = 1.0 / jnp.maximum(cs, 1.0)
    agg = t_ref[0] + t_ref[1]
    out_ref[...] = jnp.maximum(agg * invd[:, None] + z_ref[...], 0.0)


def _final(t, cnt, z):
    n = t.shape[1]
    c = t.shape[2]
    return pl.pallas_call(
        _final_body,
        grid=(n // _BLK,),
        in_specs=[
            pl.BlockSpec((_NC, _BLK, c), lambda i: (0, i, 0)),
            pl.BlockSpec((_NC, n), lambda i: (0, 0)),
            pl.BlockSpec((_BLK, c), lambda i: (i, 0)),
        ],
        out_specs=pl.BlockSpec((_BLK, c), lambda i: (i, 0)),
        out_shape=jax.ShapeDtypeStruct((n, c), jnp.float32),
    )(t, cnt, z)


def kernel(x, edge_index, W1, as1, ad1, b1, W2, as2, ad2, b2, W3, as3, ad3, b3,
           Wp, bp, Wl1, bl1, Wr1, Wl2, bl2, Wr2, Wl3, bl3, Wr3):
    src = edge_index[0]
    dst = edge_index[1]
    pad_idx = _N + (jnp.arange(_EPAD - _E, dtype=jnp.int32) % (_NTAB - _N))
    srcp = jnp.concatenate([src, pad_idx])
    dstp = jnp.concatenate([dst, pad_idx])
    x_pad = jnp.pad(x, ((0, _NPAD - _N), (0, 0)))

    y1, z1 = _proj_first(x_pad, Wl1, Wr1, bl1)
    t1, cnt = _sc_agg(y1, srcp, dstp, with_cnt=True)
    y2, z2 = _proj_mid(t1, cnt, z1, Wl2, Wr2, bl2)
    t2, = _sc_agg(y2, srcp, dstp)
    y3, z3 = _proj_mid(t2, cnt, z2, Wl3, Wr3, bl3)
    t3, = _sc_agg(y3, srcp, dstp, nbuf=6, k_ahead=4, chunk=40)
    out = _final(t3, cnt, z3)
    return out[:_N]
